# Initial kernel scaffold; baseline (speedup 1.0000x reference)
import jax
import jax.numpy as jnp
from jax.experimental import pallas as pl


def kernel(uv, x, upsample, si0, si1, si2, si3, up0_row, up0_col, up0_val, up1_row, up1_col, up1_val, up2_row, up2_col, up2_val, up3_row, up3_col, up3_val, W0, b0, W1, b1, W2, b2, W3, b3, Wh, bh):
    raise NotImplementedError("write your pallas kernel here")


# speedup vs baseline: 20.9397x; 20.9397x over previous
import functools

import jax
import jax.numpy as jnp
from jax import lax
from jax.experimental import pallas as pl
from jax.experimental.pallas import tpu as pltpu
from jax.experimental.pallas import tpu_sc as plsc

_V = (49152, 24576, 12288, 6144)
_V4 = 3072
_B = 2
_NSUB = 16
_LANES = 16
_IDX_CHUNK = 96


def _sc_mesh():
    return plsc.VectorSubcoreMesh(core_axis_name="c", subcore_axis_name="s")


def _sc_gather_rows(src, idx, n_out, c, taps, v):
    n_per_worker = (n_out // (_B * _NSUB))
    nb = taps * 64
    n_blocks = n_per_worker // nb
    n_dma = nb // _IDX_CHUNK

    @functools.partial(
        pl.kernel,
        out_type=jax.ShapeDtypeStruct((n_out, c), jnp.float32),
        mesh=_sc_mesh(),
        scratch_types=[
            pltpu.VMEM((nb,), jnp.int32),
            pltpu.VMEM((nb, c), jnp.float32),
            pltpu.SemaphoreType.DMA,
        ],
        compiler_params=pltpu.CompilerParams(use_tc_tiling_on_sc=False),
    )
    def k(src_h, idx_h, out_h, idxv, rows, sem):
        b = lax.axis_index("c")
        li = lax.axis_index("s")
        base = (b * _NSUB + li) * n_per_worker

        def blk(ib, carry):
            t0 = base + ib * nb
            pltpu.sync_copy(idx_h.at[pl.ds(t0, nb)], idxv)
            handles = []
            for i in range(n_dma):
                handles.append(
                    pltpu.async_copy(
                        src_h.at[idxv.at[pl.ds(i * _IDX_CHUNK, _IDX_CHUNK)]],
                        rows.at[pl.ds(i * _IDX_CHUNK, _IDX_CHUNK)],
                        sem,
                    ))
            for h in handles:
                h.wait()
            pltpu.sync_copy(rows, out_h.at[pl.ds(t0, nb)])
            return carry

        lax.fori_loop(0, n_blocks, blk, 0)

    return k(src, idx)


def _sc_gather9_reduce(qf, idx, bias, n_out, oc, relu):
    n_per_worker = n_out // (_B * _NSUB)
    nb = 64
    n_blocks = n_per_worker // nb
    n_rows = nb * 9
    n_dma = n_rows // _IDX_CHUNK
    n_cc = oc // _LANES

    @functools.partial(
        pl.kernel,
        out_type=jax.ShapeDtypeStruct((n_out, oc), jnp.float32),
        mesh=_sc_mesh(),
        scratch_types=[
            pltpu.VMEM((n_rows,), jnp.int32),
            pltpu.VMEM((n_rows, oc), jnp.float32),
            pltpu.VMEM((nb, oc), jnp.float32),
            pltpu.VMEM((oc,), jnp.float32),
            pltpu.SemaphoreType.DMA,
        ],
        compiler_params=pltpu.CompilerParams(use_tc_tiling_on_sc=False),
    )
    def k(qf_h, idx_h, bias_h, out_h, idxv, rows, outv, biasv, sem):
        b = lax.axis_index("c")
        li = lax.axis_index("s")
        pltpu.sync_copy(bias_h, biasv)
        base = (b * _NSUB + li) * n_per_worker

        def blk(ib, carry):
            r0 = base + ib * nb
            t0 = r0 * 9
            pltpu.sync_copy(idx_h.at[pl.ds(t0, n_rows)], idxv)
            handles = []
            for i in range(n_dma):
                handles.append(
                    pltpu.async_copy(
                        qf_h.at[idxv.at[pl.ds(i * _IDX_CHUNK, _IDX_CHUNK)]],
                        rows.at[pl.ds(i * _IDX_CHUNK, _IDX_CHUNK)],
                        sem,
                    ))
            for h in handles:
                h.wait()

            def vert(v, c2):
                r9 = v * 9
                for cc in range(n_cc):
                    sl = pl.ds(cc * _LANES, _LANES)
                    acc = biasv[sl]
                    for j in range(9):
                        acc = acc + rows[r9 + j, sl]
                    if relu:
                        acc = jnp.maximum(acc, 0.0)
                    outv[v, sl] = acc
                return c2

            lax.fori_loop(0, nb, vert, 0)
            pltpu.sync_copy(outv, out_h.at[pl.ds(r0, nb)])
            return carry

        lax.fori_loop(0, n_blocks, blk, 0)

    return k(qf, idx, bias)


def _pool_mm_body(g_ref, val_ref, w_ref, o_ref):
    g = g_ref[...]
    val = val_ref[...]
    nb = val.shape[0]
    c = g.shape[1]
    g3 = g.reshape(nb, 3, c)
    pooled = jnp.sum(g3 * val[:, :, None], axis=1)
    o_ref[...] = jnp.dot(pooled, w_ref[...], preferred_element_type=jnp.float32)


def _tc_pool_matmul(g, vals, w, bm=512):
    m = vals.shape[0]
    c, n = w.shape
    return pl.pallas_call(
        _pool_mm_body,
        grid=(m // bm,),
        in_specs=[
            pl.BlockSpec((3 * bm, c), lambda i: (i, 0)),
            pl.BlockSpec((bm, 3), lambda i: (i, 0)),
            pl.BlockSpec((c, n), lambda i: (0, 0)),
        ],
        out_specs=pl.BlockSpec((bm, n), lambda i: (i, 0)),
        out_shape=jax.ShapeDtypeStruct((m, n), jnp.float32),
    )(g, vals, w)


def _mm_body(x_ref, w_ref, o_ref):
    o_ref[...] = jnp.dot(x_ref[...], w_ref[...], preferred_element_type=jnp.float32)


def _tc_matmul(x, w, bm=512):
    m, c = x.shape
    _, n = w.shape
    return pl.pallas_call(
        _mm_body,
        grid=(m // bm,),
        in_specs=[
            pl.BlockSpec((bm, c), lambda i: (i, 0)),
            pl.BlockSpec((c, n), lambda i: (0, 0)),
        ],
        out_specs=pl.BlockSpec((bm, n), lambda i: (i, 0)),
        out_shape=jax.ShapeDtypeStruct((m, n), jnp.float32),
    )(x, w)


def _front_body(uv_ref, imt_ref, ups_ref, o_ref):
    uv = uv_ref[0]
    uvc = jnp.clip((uv - 0.5) * 2.0, -1.0, 1.0)
    gx = uvc[:, 0:1]
    gy = uvc[:, 1:2]
    xf = (gx + 1.0) * 0.5 * 31.0
    yf = (gy + 1.0) * 0.5 * 31.0
    x0 = jnp.floor(xf)
    y0 = jnp.floor(yf)
    x1 = x0 + 1.0
    y1 = y0 + 1.0
    wa = (x1 - xf) * (y1 - yf)
    wb = (x1 - xf) * (yf - y0)
    wc = (xf - x0) * (y1 - yf)
    wd = (xf - x0) * (yf - y0)
    x0c = jnp.clip(x0, 0.0, 31.0).astype(jnp.int32)
    x1c = jnp.clip(x1, 0.0, 31.0).astype(jnp.int32)
    y0c = jnp.clip(y0, 0.0, 31.0).astype(jnp.int32)
    y1c = jnp.clip(y1, 0.0, 31.0).astype(jnp.int32)
    p = lax.broadcasted_iota(jnp.int32, (256, 1024), 1)
    oh = (jnp.where(p == y0c * 32 + x0c, wa, 0.0)
          + jnp.where(p == y1c * 32 + x0c, wb, 0.0)
          + jnp.where(p == y0c * 32 + x1c, wc, 0.0)
          + jnp.where(p == y1c * 32 + x1c, wd, 0.0))
    feat = jnp.dot(oh, imt_ref[0], preferred_element_type=jnp.float32)
    o_ref[...] = jnp.dot(ups_ref[...], feat,
                         preferred_element_type=jnp.float32)[None]


def _tc_front(uv, imt, ups):
    return pl.pallas_call(
        _front_body,
        grid=(_B,),
        in_specs=[
            pl.BlockSpec((1, 256, 2), lambda b: (b, 0, 0)),
            pl.BlockSpec((1, 1024, 128), lambda b: (b, 0, 0)),
            pl.BlockSpec((_V4, 256), lambda b: (0, 0)),
        ],
        out_specs=pl.BlockSpec((1, _V4, 128), lambda b: (b, 0, 0)),
        out_shape=jax.ShapeDtypeStruct((_B, _V4, 128), jnp.float32),
    )(uv, imt, ups)


def _wcat(w, cin, cout):
    return w.reshape(9, cin, cout).transpose(1, 0, 2).reshape(cin, 9 * cout)


def _batched_col(col, vc):
    off = (jnp.arange(_B, dtype=jnp.int32) * vc)[:, None]
    return (col[None, :].astype(jnp.int32) + off).reshape(-1)


def _batched_si9(si, v):
    si9 = (si.astype(jnp.int32) * 9
           + jnp.arange(9, dtype=jnp.int32)[None, :]).reshape(-1)
    off = (jnp.arange(_B, dtype=jnp.int32) * (v * 9))[:, None]
    return (si9[None, :] + off).reshape(-1)


def _level(h, col, val, si, w, bias, vc, v, cin, oc, relu):
    g = _sc_gather_rows(h, _batched_col(col, vc), _B * v * 3, cin, 3, v)
    vals = jnp.tile(val.reshape(v, 3), (_B, 1))
    q = _tc_pool_matmul(g, vals, _wcat(w, cin, oc))
    qf = q.reshape(_B * v * 9, oc)
    return _sc_gather9_reduce(qf, _batched_si9(si, v), bias, _B * v, oc, relu)


def kernel(uv, x, upsample, si0, si1, si2, si3, up0_row, up0_col, up0_val,
           up1_row, up1_col, up1_val, up2_row, up2_col, up2_val, up3_row,
           up3_col, up3_val, W0, b0, W1, b1, W2, b2, W3, b3, Wh, bh):
    imt = x.reshape(_B, 128, 1024).transpose(0, 2, 1)
    h = _tc_front(uv, imt, upsample).reshape(_B * _V4, 128)

    h = _level(h, up3_col, up3_val, si3, W0, b0, _V4, _V[3], 128, 128, True)
    h = _level(h, up2_col, up2_val, si2, W1, b1, _V[3], _V[2], 128, 64, True)
    h = _level(h, up1_col, up1_val, si1, W2, b2, _V[2], _V[1], 64, 32, True)
    h = _level(h, up0_col, up0_val, si0, W3, b3, _V[1], _V[0], 32, 16, True)

    whp = jnp.zeros((16, 144), jnp.float32)
    whr = Wh.reshape(9, 16, 3).transpose(1, 0, 2)
    whp = whp.reshape(16, 9, 16).at[:, :, :3].set(whr).reshape(16, 144)
    bhp = jnp.zeros((16,), jnp.float32).at[:3].set(bh)
    qh = _tc_matmul(h, whp).reshape(_B * _V[0] * 9, 16)
    predp = _sc_gather9_reduce(qh, _batched_si9(si0, _V[0]), bhp,
                               _B * _V[0], 16, False)
    return predp[:, :3].reshape(_B, _V[0], 3)


# speedup vs baseline: 22.9303x; 1.0951x over previous
import functools

import jax
import jax.numpy as jnp
from jax import lax
from jax.experimental import pallas as pl
from jax.experimental.pallas import tpu as pltpu
from jax.experimental.pallas import tpu_sc as plsc

_V = (49152, 24576, 12288, 6144)
_V4 = 3072
_B = 2
_NSUB = 16
_LANES = 16
_IDX_CHUNK = 96


def _sc_mesh():
    return plsc.VectorSubcoreMesh(core_axis_name="c", subcore_axis_name="s")


def _sc_gather_rows(src, idx, n_out, c, nb=288):
    n_per_worker = (n_out // (_B * _NSUB))
    n2 = n_per_worker // (2 * nb)
    n_dma = nb // _IDX_CHUNK

    @functools.partial(
        pl.kernel,
        out_type=jax.ShapeDtypeStruct((n_out, c), jnp.float32),
        mesh=_sc_mesh(),
        scratch_types=[
            pltpu.VMEM((nb,), jnp.int32),
            pltpu.VMEM((nb,), jnp.int32),
            pltpu.VMEM((nb, c), jnp.float32),
            pltpu.VMEM((nb, c), jnp.float32),
            pltpu.SemaphoreType.DMA,
            pltpu.SemaphoreType.DMA,
        ],
        compiler_params=pltpu.CompilerParams(use_tc_tiling_on_sc=False),
    )
    def k(src_h, idx_h, out_h, idxv0, idxv1, rows0, rows1, semg, semw):
        b = lax.axis_index("c")
        li = lax.axis_index("s")
        base = (b * _NSUB + li) * n_per_worker

        def fire(t0, idxv, rows):
            pltpu.sync_copy(idx_h.at[pl.ds(t0, nb)], idxv)
            hs = []
            for i in range(n_dma):
                hs.append(
                    pltpu.async_copy(
                        src_h.at[idxv.at[pl.ds(i * _IDX_CHUNK, _IDX_CHUNK)]],
                        rows.at[pl.ds(i * _IDX_CHUNK, _IDX_CHUNK)],
                        semg,
                    ))
            return hs

        def blk(ih, carry):
            t0 = base + ih * (2 * nb)
            t1 = t0 + nb
            ha = fire(t0, idxv0, rows0)
            hb = fire(t1, idxv1, rows1)
            for h in ha:
                h.wait()
            wa = pltpu.async_copy(rows0, out_h.at[pl.ds(t0, nb)], semw)
            for h in hb:
                h.wait()
            wb = pltpu.async_copy(rows1, out_h.at[pl.ds(t1, nb)], semw)
            wa.wait()
            wb.wait()
            return carry

        lax.fori_loop(0, n2, blk, 0)

    return k(src, idx)


def _sc_gather9_reduce(qf, idx, bias, n_out, oc, relu, nb):
    n_per_worker = n_out // (_B * _NSUB)
    n2 = n_per_worker // (2 * nb)
    n_rows = nb * 9
    n_dma = n_rows // _IDX_CHUNK
    n_cc = oc // _LANES
    unroll = 4

    @functools.partial(
        pl.kernel,
        out_type=jax.ShapeDtypeStruct((n_out, oc), jnp.float32),
        mesh=_sc_mesh(),
        scratch_types=[
            pltpu.VMEM((n_rows,), jnp.int32),
            pltpu.VMEM((n_rows,), jnp.int32),
            pltpu.VMEM((n_rows, oc), jnp.float32),
            pltpu.VMEM((n_rows, oc), jnp.float32),
            pltpu.VMEM((nb, oc), jnp.float32),
            pltpu.VMEM((oc,), jnp.float32),
            pltpu.SemaphoreType.DMA,
        ],
        compiler_params=pltpu.CompilerParams(use_tc_tiling_on_sc=False),
    )
    def k(qf_h, idx_h, bias_h, out_h, idxv0, idxv1, rows0, rows1, outv,
          biasv, sem):
        b = lax.axis_index("c")
        li = lax.axis_index("s")
        pltpu.sync_copy(bias_h, biasv)
        base = (b * _NSUB + li) * n_per_worker

        def fire(r0, idxv, rows):
            pltpu.sync_copy(idx_h.at[pl.ds(r0 * 9, n_rows)], idxv)
            hs = []
            for i in range(n_dma):
                hs.append(
                    pltpu.async_copy(
                        qf_h.at[idxv.at[pl.ds(i * _IDX_CHUNK, _IDX_CHUNK)]],
                        rows.at[pl.ds(i * _IDX_CHUNK, _IDX_CHUNK)],
                        sem,
                    ))
            return hs

        def reduce_block(r0, rows):
            def vert(v4, c2):
                for u in range(unroll):
                    v = v4 * unroll + u
                    r9 = v * 9
                    for cc in range(n_cc):
                        sl = pl.ds(cc * _LANES, _LANES)
                        acc = biasv[sl]
                        for j in range(9):
                            acc = acc + rows[r9 + j, sl]
                        if relu:
                            acc = jnp.maximum(acc, 0.0)
                        outv[v, sl] = acc
                return c2

            lax.fori_loop(0, nb // unroll, vert, 0)
            pltpu.sync_copy(outv, out_h.at[pl.ds(r0, nb)])

        def blk(ih, carry):
            r0 = base + ih * (2 * nb)
            r1 = r0 + nb
            ha = fire(r0, idxv0, rows0)
            hb = fire(r1, idxv1, rows1)
            for h in ha:
                h.wait()
            reduce_block(r0, rows0)
            for h in hb:
                h.wait()
            reduce_block(r1, rows1)
            return carry

        lax.fori_loop(0, n2, blk, 0)

    return k(qf, idx, bias)


def _pool_mm_body(g_ref, val_ref, w_ref, o_ref):
    g = g_ref[...]
    val = val_ref[...]
    nb = val.shape[0]
    c = g.shape[1]
    g3 = g.reshape(nb, 3, c)
    pooled = jnp.sum(g3 * val[:, :, None], axis=1)
    o_ref[...] = jnp.dot(pooled, w_ref[...], preferred_element_type=jnp.float32)


def _tc_pool_matmul(g, vals, w, bm=512):
    m = vals.shape[0]
    c, n = w.shape
    return pl.pallas_call(
        _pool_mm_body,
        grid=(m // bm,),
        in_specs=[
            pl.BlockSpec((3 * bm, c), lambda i: (i, 0)),
            pl.BlockSpec((bm, 3), lambda i: (i, 0)),
            pl.BlockSpec((c, n), lambda i: (0, 0)),
        ],
        out_specs=pl.BlockSpec((bm, n), lambda i: (i, 0)),
        out_shape=jax.ShapeDtypeStruct((m, n), jnp.float32),
    )(g, vals, w)


def _mm_body(x_ref, w_ref, o_ref):
    o_ref[...] = jnp.dot(x_ref[...], w_ref[...], preferred_element_type=jnp.float32)


def _tc_matmul(x, w, bm=512):
    m, c = x.shape
    _, n = w.shape
    return pl.pallas_call(
        _mm_body,
        grid=(m // bm,),
        in_specs=[
            pl.BlockSpec((bm, c), lambda i: (i, 0)),
            pl.BlockSpec((c, n), lambda i: (0, 0)),
        ],
        out_specs=pl.BlockSpec((bm, n), lambda i: (i, 0)),
        out_shape=jax.ShapeDtypeStruct((m, n), jnp.float32),
    )(x, w)


def _front_body(uv_ref, imt_ref, ups_ref, o_ref):
    uv = uv_ref[0]
    uvc = jnp.clip((uv - 0.5) * 2.0, -1.0, 1.0)
    gx = uvc[:, 0:1]
    gy = uvc[:, 1:2]
    xf = (gx + 1.0) * 0.5 * 31.0
    yf = (gy + 1.0) * 0.5 * 31.0
    x0 = jnp.floor(xf)
    y0 = jnp.floor(yf)
    x1 = x0 + 1.0
    y1 = y0 + 1.0
    wa = (x1 - xf) * (y1 - yf)
    wb = (x1 - xf) * (yf - y0)
    wc = (xf - x0) * (y1 - yf)
    wd = (xf - x0) * (yf - y0)
    x0c = jnp.clip(x0, 0.0, 31.0).astype(jnp.int32)
    x1c = jnp.clip(x1, 0.0, 31.0).astype(jnp.int32)
    y0c = jnp.clip(y0, 0.0, 31.0).astype(jnp.int32)
    y1c = jnp.clip(y1, 0.0, 31.0).astype(jnp.int32)
    p = lax.broadcasted_iota(jnp.int32, (256, 1024), 1)
    oh = (jnp.where(p == y0c * 32 + x0c, wa, 0.0)
          + jnp.where(p == y1c * 32 + x0c, wb, 0.0)
          + jnp.where(p == y0c * 32 + x1c, wc, 0.0)
          + jnp.where(p == y1c * 32 + x1c, wd, 0.0))
    feat = jnp.dot(oh, imt_ref[0], preferred_element_type=jnp.float32)
    o_ref[...] = jnp.dot(ups_ref[...], feat,
                         preferred_element_type=jnp.float32)[None]


def _tc_front(uv, imt, ups):
    return pl.pallas_call(
        _front_body,
        grid=(_B,),
        in_specs=[
            pl.BlockSpec((1, 256, 2), lambda b: (b, 0, 0)),
            pl.BlockSpec((1, 1024, 128), lambda b: (b, 0, 0)),
            pl.BlockSpec((_V4, 256), lambda b: (0, 0)),
        ],
        out_specs=pl.BlockSpec((1, _V4, 128), lambda b: (b, 0, 0)),
        out_shape=jax.ShapeDtypeStruct((_B, _V4, 128), jnp.float32),
    )(uv, imt, ups)


def _wcat(w, cin, cout):
    return w.reshape(9, cin, cout).transpose(1, 0, 2).reshape(cin, 9 * cout)


def _batched_col(col, vc):
    off = (jnp.arange(_B, dtype=jnp.int32) * vc)[:, None]
    return (col[None, :].astype(jnp.int32) + off).reshape(-1)


def _batched_si9(si, v):
    si9 = (si.astype(jnp.int32) * 9
           + jnp.arange(9, dtype=jnp.int32)[None, :]).reshape(-1)
    off = (jnp.arange(_B, dtype=jnp.int32) * (v * 9))[:, None]
    return (si9[None, :] + off).reshape(-1)


_G9_NB = {128: 32, 64: 64, 32: 96, 16: 128}


def _level(h, col, val, si, w, bias, vc, v, cin, oc, relu):
    g = _sc_gather_rows(h, _batched_col(col, vc), _B * v * 3, cin)
    vals = jnp.tile(val.reshape(v, 3), (_B, 1))
    q = _tc_pool_matmul(g, vals, _wcat(w, cin, oc))
    qf = q.reshape(_B * v * 9, oc)
    return _sc_gather9_reduce(qf, _batched_si9(si, v), bias, _B * v, oc, relu,
                              _G9_NB[oc])


def kernel(uv, x, upsample, si0, si1, si2, si3, up0_row, up0_col, up0_val,
           up1_row, up1_col, up1_val, up2_row, up2_col, up2_val, up3_row,
           up3_col, up3_val, W0, b0, W1, b1, W2, b2, W3, b3, Wh, bh):
    imt = x.reshape(_B, 128, 1024).transpose(0, 2, 1)
    h = _tc_front(uv, imt, upsample).reshape(_B * _V4, 128)

    h = _level(h, up3_col, up3_val, si3, W0, b0, _V4, _V[3], 128, 128, True)
    h = _level(h, up2_col, up2_val, si2, W1, b1, _V[3], _V[2], 128, 64, True)
    h = _level(h, up1_col, up1_val, si1, W2, b2, _V[2], _V[1], 64, 32, True)
    h = _level(h, up0_col, up0_val, si0, W3, b3, _V[1], _V[0], 32, 16, True)

    whp = jnp.zeros((16, 144), jnp.float32)
    whr = Wh.reshape(9, 16, 3).transpose(1, 0, 2)
    whp = whp.reshape(16, 9, 16).at[:, :, :3].set(whr).reshape(16, 144)
    bhp = jnp.zeros((16,), jnp.float32).at[:3].set(bh)
    qh = _tc_matmul(h, whp).reshape(_B * _V[0] * 9, 16)
    predp = _sc_gather9_reduce(qh, _batched_si9(si0, _V[0]), bhp,
                               _B * _V[0], 16, False, _G9_NB[16])
    return predp[:, :3].reshape(_B, _V[0], 3)
